# baseline (speedup 1.0000x reference)
import functools

import jax
import jax.numpy as jnp
from jax import lax
from jax.experimental import pallas as pl
from jax.experimental.pallas import tpu as pltpu
from jax.experimental.pallas import tpu_sc as plsc

EMBED_DIM = 32
NUM_CORES = 2
NUM_SUBCORES = 16
NUM_WORKERS = NUM_CORES * NUM_SUBCORES
LANES = 16
ROW_UNROLL = 1


@functools.lru_cache(maxsize=None)
def _build_mf_kernel(batch: int):
    assert batch % (NUM_WORKERS * LANES) == 0
    b_per_w = batch // NUM_WORKERS
    mesh = plsc.VectorSubcoreMesh(
        core_axis_name="c", subcore_axis_name="s", num_cores=NUM_CORES
    )

    @functools.partial(
        pl.kernel,
        mesh=mesh,
        compiler_params=pltpu.CompilerParams(
            needs_layout_passes=False, use_tc_tiling_on_sc=False
        ),
        out_type=jax.ShapeDtypeStruct((batch,), jnp.float32),
        scratch_types=[
            pltpu.VMEM((2, b_per_w), jnp.int32),
            pltpu.VMEM((b_per_w, EMBED_DIM), jnp.float32),
            pltpu.VMEM((b_per_w, EMBED_DIM), jnp.float32),
            pltpu.VMEM((b_per_w,), jnp.float32),
            pltpu.VMEM((b_per_w,), jnp.float32),
            pltpu.VMEM((LANES,), jnp.float32),
            pltpu.VMEM((LANES * b_per_w,), jnp.float32),
            pltpu.VMEM((b_per_w,), jnp.float32),
            pltpu.SemaphoreType.DMA,
            pltpu.SemaphoreType.DMA,
            pltpu.SemaphoreType.DMA,
        ],
    )
    def mf_kernel(
        ids_hbm, utab_hbm, itab_hbm, ubias_hbm, ibias_hbm, gb_hbm,
        out_hbm,
        idx_v, urows_v, irows_v, ubias_v, ibias_v, gb_v, pt_v, out_v,
        sem, sem2, bias_sem,
    ):
        wid = lax.axis_index("s") * NUM_CORES + lax.axis_index("c")
        base = wid * b_per_w

        half = b_per_w // 2
        pltpu.sync_copy(ids_hbm.at[wid, :, pl.ds(0, half)],
                        idx_v.at[:, pl.ds(0, half)])
        row_copies0 = (
            pltpu.async_copy(utab_hbm.at[idx_v.at[0, pl.ds(0, half)]],
                             urows_v.at[pl.ds(0, half), :], sem),
            pltpu.async_copy(itab_hbm.at[idx_v.at[1, pl.ds(0, half)]],
                             irows_v.at[pl.ds(0, half), :], sem),
        )
        pltpu.sync_copy(ids_hbm.at[wid, :, pl.ds(half, half)],
                        idx_v.at[:, pl.ds(half, half)])
        row_copies1 = (
            pltpu.async_copy(utab_hbm.at[idx_v.at[0, pl.ds(half, half)]],
                             urows_v.at[pl.ds(half, half), :], sem2),
            pltpu.async_copy(itab_hbm.at[idx_v.at[1, pl.ds(half, half)]],
                             irows_v.at[pl.ds(half, half), :], sem2),
        )
        bias_copies = (
            pltpu.async_copy(ubias_hbm.at[idx_v.at[0]], ubias_v, bias_sem),
            pltpu.async_copy(ibias_hbm.at[idx_v.at[1]], ibias_v, bias_sem),
        )
        pltpu.sync_copy(gb_hbm, gb_v)

        lane_off = lax.iota(jnp.int32, LANES) * b_per_w

        for cp in row_copies0:
            cp.wait()

        @plsc.parallel_loop(0, half, ROW_UNROLL)
        def dot_body0(r):
            for u in range(ROW_UNROLL):
                ru = r + u
                prod = (
                    urows_v[ru, pl.ds(0, LANES)]
                    * irows_v[ru, pl.ds(0, LANES)]
                    + urows_v[ru, pl.ds(LANES, LANES)]
                    * irows_v[ru, pl.ds(LANES, LANES)]
                )
                plsc.store_scatter(pt_v, [lane_off + ru], prod)

        for cp in row_copies1:
            cp.wait()

        @plsc.parallel_loop(half, b_per_w, ROW_UNROLL)
        def dot_body1(r):
            for u in range(ROW_UNROLL):
                ru = r + u
                prod = (
                    urows_v[ru, pl.ds(0, LANES)]
                    * irows_v[ru, pl.ds(0, LANES)]
                    + urows_v[ru, pl.ds(LANES, LANES)]
                    * irows_v[ru, pl.ds(LANES, LANES)]
                )
                plsc.store_scatter(pt_v, [lane_off + ru], prod)

        for cp in bias_copies:
            cp.wait()
        gb = gb_v[...]

        @plsc.parallel_loop(0, b_per_w, LANES)
        def fold_body(off):
            acc = ubias_v[pl.ds(off, LANES)] + ibias_v[pl.ds(off, LANES)] + gb
            for l in range(LANES):
                acc = acc + pt_v[pl.ds(l * b_per_w + off, LANES)]
            out_v[pl.ds(off, LANES)] = acc

        pltpu.sync_copy(out_v, out_hbm.at[pl.ds(base, b_per_w)])

    return mf_kernel


def kernel(ids, embedding_users, embedding_items, bias_users, bias_items,
           global_bias):
    batch = ids.shape[0]
    b_per_w = batch // NUM_WORKERS
    idall = (
        ids.astype(jnp.int32).reshape(NUM_WORKERS, b_per_w, 2)
        .transpose(0, 2, 1)
    )
    utab = embedding_users.reshape(-1, EMBED_DIM)
    itab = embedding_items.reshape(-1, EMBED_DIM)
    gb = jnp.broadcast_to(global_bias.astype(jnp.float32), (LANES,))
    return _build_mf_kernel(batch)(
        idall, utab, itab, bias_users, bias_items, gb
    )


# speedup vs baseline: 1.0186x; 1.0186x over previous
import functools

import jax
import jax.numpy as jnp
from jax import lax
from jax.experimental import pallas as pl
from jax.experimental.pallas import tpu as pltpu
from jax.experimental.pallas import tpu_sc as plsc

EMBED_DIM = 32
NUM_CORES = 2
NUM_SUBCORES = 16
NUM_WORKERS = NUM_CORES * NUM_SUBCORES
LANES = 16
ROW_UNROLL = 2


@functools.lru_cache(maxsize=None)
def _build_mf_kernel(batch: int):
    assert batch % (NUM_WORKERS * LANES) == 0
    b_per_w = batch // NUM_WORKERS
    mesh = plsc.VectorSubcoreMesh(
        core_axis_name="c", subcore_axis_name="s", num_cores=NUM_CORES
    )

    @functools.partial(
        pl.kernel,
        mesh=mesh,
        compiler_params=pltpu.CompilerParams(
            needs_layout_passes=False, use_tc_tiling_on_sc=False
        ),
        out_type=jax.ShapeDtypeStruct((batch,), jnp.float32),
        scratch_types=[
            pltpu.VMEM((2, b_per_w), jnp.int32),
            pltpu.VMEM((b_per_w, EMBED_DIM), jnp.float32),
            pltpu.VMEM((b_per_w, EMBED_DIM), jnp.float32),
            pltpu.VMEM((b_per_w,), jnp.float32),
            pltpu.VMEM((b_per_w,), jnp.float32),
            pltpu.VMEM((LANES,), jnp.float32),
            pltpu.VMEM((LANES * b_per_w,), jnp.float32),
            pltpu.VMEM((b_per_w,), jnp.float32),
            pltpu.SemaphoreType.DMA,
            pltpu.SemaphoreType.DMA,
            pltpu.SemaphoreType.DMA,
        ],
    )
    def mf_kernel(
        ids_hbm, utab_hbm, itab_hbm, ubias_hbm, ibias_hbm, gb_hbm,
        out_hbm,
        idx_v, urows_v, irows_v, ubias_v, ibias_v, gb_v, pt_v, out_v,
        sem, sem2, bias_sem,
    ):
        wid = lax.axis_index("s") * NUM_CORES + lax.axis_index("c")
        base = wid * b_per_w

        half = b_per_w // 2
        pltpu.sync_copy(ids_hbm.at[wid, :, pl.ds(0, half)],
                        idx_v.at[:, pl.ds(0, half)])
        row_copies0 = (
            pltpu.async_copy(utab_hbm.at[idx_v.at[0, pl.ds(0, half)]],
                             urows_v.at[pl.ds(0, half), :], sem),
            pltpu.async_copy(itab_hbm.at[idx_v.at[1, pl.ds(0, half)]],
                             irows_v.at[pl.ds(0, half), :], sem),
        )
        pltpu.sync_copy(ids_hbm.at[wid, :, pl.ds(half, half)],
                        idx_v.at[:, pl.ds(half, half)])
        row_copies1 = (
            pltpu.async_copy(utab_hbm.at[idx_v.at[0, pl.ds(half, half)]],
                             urows_v.at[pl.ds(half, half), :], sem2),
            pltpu.async_copy(itab_hbm.at[idx_v.at[1, pl.ds(half, half)]],
                             irows_v.at[pl.ds(half, half), :], sem2),
        )
        bias_copies = (
            pltpu.async_copy(ubias_hbm.at[idx_v.at[0]], ubias_v, bias_sem),
            pltpu.async_copy(ibias_hbm.at[idx_v.at[1]], ibias_v, bias_sem),
        )
        pltpu.sync_copy(gb_hbm, gb_v)

        lane_off = lax.iota(jnp.int32, LANES) * b_per_w

        for cp in row_copies0:
            cp.wait()

        @plsc.parallel_loop(0, half, ROW_UNROLL)
        def dot_body0(r):
            for u in range(ROW_UNROLL):
                ru = r + u
                prod = (
                    urows_v[ru, pl.ds(0, LANES)]
                    * irows_v[ru, pl.ds(0, LANES)]
                    + urows_v[ru, pl.ds(LANES, LANES)]
                    * irows_v[ru, pl.ds(LANES, LANES)]
                )
                plsc.store_scatter(pt_v, [lane_off + ru], prod)

        for cp in row_copies1:
            cp.wait()

        @plsc.parallel_loop(half, b_per_w, ROW_UNROLL)
        def dot_body1(r):
            for u in range(ROW_UNROLL):
                ru = r + u
                prod = (
                    urows_v[ru, pl.ds(0, LANES)]
                    * irows_v[ru, pl.ds(0, LANES)]
                    + urows_v[ru, pl.ds(LANES, LANES)]
                    * irows_v[ru, pl.ds(LANES, LANES)]
                )
                plsc.store_scatter(pt_v, [lane_off + ru], prod)

        for cp in bias_copies:
            cp.wait()
        gb = gb_v[...]

        @plsc.parallel_loop(0, b_per_w, LANES)
        def fold_body(off):
            acc = ubias_v[pl.ds(off, LANES)] + ibias_v[pl.ds(off, LANES)] + gb
            for l in range(LANES):
                acc = acc + pt_v[pl.ds(l * b_per_w + off, LANES)]
            out_v[pl.ds(off, LANES)] = acc

        pltpu.sync_copy(out_v, out_hbm.at[pl.ds(base, b_per_w)])

    return mf_kernel


def kernel(ids, embedding_users, embedding_items, bias_users, bias_items,
           global_bias):
    batch = ids.shape[0]
    b_per_w = batch // NUM_WORKERS
    idall = (
        ids.astype(jnp.int32).reshape(NUM_WORKERS, b_per_w, 2)
        .transpose(0, 2, 1)
    )
    utab = embedding_users.reshape(-1, EMBED_DIM)
    itab = embedding_items.reshape(-1, EMBED_DIM)
    gb = jnp.broadcast_to(global_bias.astype(jnp.float32), (LANES,))
    return _build_mf_kernel(batch)(
        idall, utab, itab, bias_users, bias_items, gb
    )
